# Initial kernel scaffold; baseline (speedup 1.0000x reference)
import jax
import jax.numpy as jnp
from jax.experimental import pallas as pl


def kernel(input, offsets, weight, bias):
    raise NotImplementedError("write your pallas kernel here")


# speedup vs baseline: 5.9064x; 5.9064x over previous
import functools

import jax
import jax.numpy as jnp
from jax.experimental import pallas as pl
from jax.experimental.pallas import tpu as pltpu

N = 32768
B = 16
D = 1024
EPS = 1e-05

ROWS = 1024


def _onehot(off_ref, step, rows):
    r = step * rows + jax.lax.broadcasted_iota(jnp.int32, (rows, 1), 0)
    cols = []
    for b in range(B):
        start = off_ref[b - 1] if b > 0 else 0
        end = off_ref[b]
        cols.append(((r >= start) & (r < end)).astype(jnp.float32))
    return jnp.concatenate(cols, axis=1)


def _stats_kernel(off_ref, x_ref, sum_ref, sq_ref):
    step = pl.program_id(0)
    oh = _onehot(off_ref, step, ROWS)
    x = x_ref[...]
    dims = (((0,), (0,)), ((), ()))
    ps = jax.lax.dot_general(oh, x, dims, preferred_element_type=jnp.float32)
    psq = jax.lax.dot_general(oh, x * x, dims,
                              preferred_element_type=jnp.float32)

    @pl.when(step == 0)
    def _():
        sum_ref[...] = ps
        sq_ref[...] = psq

    @pl.when(step != 0)
    def _():
        sum_ref[...] += ps
        sq_ref[...] += psq


def _norm_kernel(off_ref, x_ref, sum_ref, sq_ref, w_ref, b_ref, out_ref):
    step = pl.program_id(0)
    lens = []
    for b in range(B):
        start = off_ref[b - 1] if b > 0 else 0
        lens.append(jnp.maximum(off_ref[b] - start, 1))
    inv_cnt = 1.0 / jnp.stack(lens).astype(jnp.float32).reshape(B, 1)
    s = sum_ref[...]
    sq = sq_ref[...]
    mean = s * inv_cnt
    var = sq * inv_cnt - mean * mean
    rstd = jax.lax.rsqrt(jnp.maximum(var, 0.0) + EPS)
    scale = rstd * w_ref[...]
    shift = b_ref[...] - mean * scale
    oh = _onehot(off_ref, step, ROWS)
    dims = (((1,), (0,)), ((), ()))
    row_scale = jax.lax.dot_general(oh, scale, dims,
                                    preferred_element_type=jnp.float32)
    row_shift = jax.lax.dot_general(oh, shift, dims,
                                    preferred_element_type=jnp.float32)
    out_ref[...] = x_ref[...] * row_scale + row_shift


@functools.partial(jax.jit, static_argnames=("interpret",))
def kernel(input, offsets, weight, bias, interpret=False):
    steps = N // ROWS
    stats_grid = pltpu.PrefetchScalarGridSpec(
        num_scalar_prefetch=1,
        grid=(steps,),
        in_specs=[pl.BlockSpec((ROWS, D), lambda i, off: (i, 0))],
        out_specs=[pl.BlockSpec((B, D), lambda i, off: (0, 0)),
                   pl.BlockSpec((B, D), lambda i, off: (0, 0))],
    )
    ssum, ssq = pl.pallas_call(
        _stats_kernel,
        grid_spec=stats_grid,
        out_shape=[jax.ShapeDtypeStruct((B, D), jnp.float32),
                   jax.ShapeDtypeStruct((B, D), jnp.float32)],
        interpret=interpret,
    )(offsets, input)

    norm_grid = pltpu.PrefetchScalarGridSpec(
        num_scalar_prefetch=1,
        grid=(steps,),
        in_specs=[pl.BlockSpec((ROWS, D), lambda i, off: (i, 0)),
                  pl.BlockSpec((B, D), lambda i, off: (0, 0)),
                  pl.BlockSpec((B, D), lambda i, off: (0, 0)),
                  pl.BlockSpec((1, D), lambda i, off: (0, 0)),
                  pl.BlockSpec((1, D), lambda i, off: (0, 0))],
        out_specs=pl.BlockSpec((ROWS, D), lambda i, off: (i, 0)),
    )
    out = pl.pallas_call(
        _norm_kernel,
        grid_spec=norm_grid,
        out_shape=jax.ShapeDtypeStruct((N, D), jnp.float32),
        interpret=interpret,
    )(offsets, input, ssum, ssq,
      weight.reshape(1, D), bias.reshape(1, D))
    return out
